# baseline (speedup 1.0000x reference)
import functools

import jax
import jax.numpy as jnp
from jax import lax
from jax.experimental import pallas as pl
from jax.experimental.pallas import tpu as pltpu
from jax.experimental.pallas import tpu_sc as plsc

N, E, D, DE = 10000, 320000, 128, 16

NC, NS = 2, 16
NW = NC * NS

EPS = 128
RPG = 16
B_EDGE = EPS * RPG
GROUPS = 5
E_PAD = NW * GROUPS * B_EDGE
ROWS_TOTAL = E_PAD // EPS
ROWS_PER_WORKER = ROWS_TOTAL // NW
N_PAD = 10240
NODE_SLICE = N_PAD // NS

_mesh = plsc.VectorSubcoreMesh(
    core_axis_name="c", subcore_axis_name="s", num_cores=NC, num_subcores=NS
)


@functools.partial(
    pl.kernel,
    out_type=jax.ShapeDtypeStruct((NC * N_PAD, DE), jnp.float32),
    mesh=_mesh,
    scratch_types=[
        pltpu.VMEM((ROWS_PER_WORKER, EPS), jnp.int32),
        pltpu.VMEM((ROWS_PER_WORKER, EPS), jnp.int32),
        pltpu.VMEM((ROWS_PER_WORKER * EPS,), jnp.float32),
        pltpu.VMEM((B_EDGE, DE), jnp.float32),
        pltpu.VMEM((B_EDGE, DE), jnp.float32),
        pltpu.VMEM_SHARED((N_PAD, DE), jnp.float32),
        pltpu.VMEM_SHARED((N_PAD, DE), jnp.float32),
        pltpu.SemaphoreType.DMA,
        pltpu.SemaphoreType.DMA,
        pltpu.SemaphoreType.DMA,
        pltpu.SemaphoreType.DMA,
    ],
    compiler_params=pltpu.CompilerParams(use_tc_tiling_on_sc=False),
)
def _edge_pass(src_hbm, dst_hbm, norm_hbm, g_hbm, zeros_hbm, agg_out,
               idx_s, idx_d, norm_v, rows0, rows1, agg_sh, g_sh,
               sem0, sem1, ssem0, ssem1):
    c = lax.axis_index("c")
    s = lax.axis_index("s")
    wid = s * NC + c

    row0 = wid * ROWS_PER_WORKER
    init_copies = [
        pltpu.async_copy(
            zeros_hbm.at[pl.ds(s * NODE_SLICE, NODE_SLICE)],
            agg_sh.at[pl.ds(s * NODE_SLICE, NODE_SLICE)], sem0),
        pltpu.async_copy(
            g_hbm.at[pl.ds(s * NODE_SLICE, NODE_SLICE)],
            g_sh.at[pl.ds(s * NODE_SLICE, NODE_SLICE)], sem0),
        pltpu.async_copy(src_hbm.at[pl.ds(row0, ROWS_PER_WORKER)],
                         idx_s, sem1),
        pltpu.async_copy(dst_hbm.at[pl.ds(row0, ROWS_PER_WORKER)],
                         idx_d, sem1),
        pltpu.async_copy(norm_hbm.at[pl.ds(row0 * EPS,
                                           ROWS_PER_WORKER * EPS)],
                         norm_v, sem1),
    ]
    for cp in init_copies:
        cp.wait()
    plsc.subcore_barrier()

    bufs = (rows0, rows1)
    gsems = (sem0, sem1)
    ssems = (ssem0, ssem1)

    def stage(b, gi):
        return [
            pltpu.async_copy(g_sh.at[idx_s.at[gi * RPG + j]],
                             bufs[b].at[pl.ds(j * EPS, EPS)], gsems[b])
            for j in range(RPG)
        ]

    inflight = {0: stage(0, 0)}
    scat = {0: [], 1: []}
    for gi in range(GROUPS):
        b = gi & 1
        if gi + 1 < GROUPS:
            for cp in scat[1 - b]:
                cp.wait()
            scat[1 - b] = []
            inflight[1 - b] = stage(1 - b, gi + 1)
        for cp in inflight[b]:
            cp.wait()

        rows_v = bufs[b]
        nbase = gi * B_EDGE

        def chunk_body(m, _, rows_v=rows_v, nbase=nbase):
            nv = norm_v[pl.ds(nbase + m * DE, DE)]
            base = m * DE
            for i in range(DE):
                rows_v[base + i] = rows_v[base + i] * nv[i]
            return 0

        lax.fori_loop(0, B_EDGE // DE, chunk_body, 0, unroll=8)

        scat[b] = [
            pltpu.async_copy(rows_v.at[pl.ds(j * EPS, EPS)],
                             agg_sh.at[idx_d.at[gi * RPG + j]], ssems[b],
                             add=True)
            for j in range(RPG)
        ]

    for b in (0, 1):
        for cp in scat[b]:
            cp.wait()

    plsc.subcore_barrier()
    pltpu.sync_copy(
        agg_sh.at[pl.ds(s * NODE_SLICE, NODE_SLICE)],
        agg_out.at[pl.ds(c * N_PAD + s * NODE_SLICE, NODE_SLICE)],
    )


_ROWS_BLK = 1000
_GRID = N // _ROWS_BLK


def _tc_a_body(z_ref, wm_ref, bm_ref, ws_ref, g0_ref, s0_ref):
    z = z_ref[...]
    g0_ref[...] = jnp.maximum(
        jnp.dot(z, wm_ref[...], preferred_element_type=jnp.float32) + bm_ref[...], 0.0)
    s0_ref[...] = jnp.dot(z, ws_ref[...], preferred_element_type=jnp.float32)


_tc_a = pl.pallas_call(
    _tc_a_body,
    grid=(_GRID,),
    in_specs=[
        pl.BlockSpec((_ROWS_BLK, D), lambda i: (i, 0)),
        pl.BlockSpec((D, DE), lambda i: (0, 0)),
        pl.BlockSpec((1, DE), lambda i: (0, 0)),
        pl.BlockSpec((D, D), lambda i: (0, 0)),
    ],
    out_specs=[
        pl.BlockSpec((_ROWS_BLK, DE), lambda i: (i, 0)),
        pl.BlockSpec((_ROWS_BLK, D), lambda i: (i, 0)),
    ],
    out_shape=[
        jax.ShapeDtypeStruct((N, DE), jnp.float32),
        jax.ShapeDtypeStruct((N, D), jnp.float32),
    ],
)


def _tc_b_body(s0_ref, a0_ref, a1_ref, wagg_ref, bnode_ref, wmsg_ref, we_ref,
               bmsg_ref, g0_ref, wself_ref, g1_ref, s1_ref):
    agg = a0_ref[...] + a1_ref[...]
    x1 = jnp.maximum(
        s0_ref[...]
        + jnp.dot(agg, wagg_ref[...], preferred_element_type=jnp.float32)
        + bnode_ref[...], 0.0)
    g1_ref[...] = jnp.maximum(
        jnp.dot(x1, wmsg_ref[...], preferred_element_type=jnp.float32)
        + jnp.dot(g0_ref[...], we_ref[...], preferred_element_type=jnp.float32)
        + bmsg_ref[...], 0.0)
    s1_ref[...] = jnp.dot(x1, wself_ref[...], preferred_element_type=jnp.float32)


_tc_b = pl.pallas_call(
    _tc_b_body,
    grid=(_GRID,),
    in_specs=[
        pl.BlockSpec((_ROWS_BLK, D), lambda i: (i, 0)),
        pl.BlockSpec((_ROWS_BLK, DE), lambda i: (i, 0)),
        pl.BlockSpec((_ROWS_BLK, DE), lambda i: (i, 0)),
        pl.BlockSpec((DE, D), lambda i: (0, 0)),
        pl.BlockSpec((1, D), lambda i: (0, 0)),
        pl.BlockSpec((D, DE), lambda i: (0, 0)),
        pl.BlockSpec((DE, DE), lambda i: (0, 0)),
        pl.BlockSpec((1, DE), lambda i: (0, 0)),
        pl.BlockSpec((_ROWS_BLK, DE), lambda i: (i, 0)),
        pl.BlockSpec((D, D), lambda i: (0, 0)),
    ],
    out_specs=[
        pl.BlockSpec((_ROWS_BLK, DE), lambda i: (i, 0)),
        pl.BlockSpec((_ROWS_BLK, D), lambda i: (i, 0)),
    ],
    out_shape=[
        jax.ShapeDtypeStruct((N, DE), jnp.float32),
        jax.ShapeDtypeStruct((N, D), jnp.float32),
    ],
)


def _tc_c_body(s1_ref, a0_ref, a1_ref, wagg_ref, bnode_ref, out_ref):
    agg = a0_ref[...] + a1_ref[...]
    out_ref[...] = jnp.maximum(
        s1_ref[...]
        + jnp.dot(agg, wagg_ref[...], preferred_element_type=jnp.float32)
        + bnode_ref[...], 0.0)


_tc_c = pl.pallas_call(
    _tc_c_body,
    grid=(_GRID,),
    in_specs=[
        pl.BlockSpec((_ROWS_BLK, D), lambda i: (i, 0)),
        pl.BlockSpec((_ROWS_BLK, DE), lambda i: (i, 0)),
        pl.BlockSpec((_ROWS_BLK, DE), lambda i: (i, 0)),
        pl.BlockSpec((DE, D), lambda i: (0, 0)),
        pl.BlockSpec((1, D), lambda i: (0, 0)),
    ],
    out_specs=pl.BlockSpec((_ROWS_BLK, D), lambda i: (i, 0)),
    out_shape=jax.ShapeDtypeStruct((N, D), jnp.float32),
)


def kernel(edge_index, z, norm, W_msg0, b_msg0, W_self0, W_agg0, b_node0,
           W_msg1, W_e1, b_msg1, W_self1, W_agg1, b_node1):
    pad = E_PAD - E
    src = jnp.concatenate(
        [edge_index[0].astype(jnp.int32), jnp.zeros((pad,), jnp.int32)]
    ).reshape(ROWS_TOTAL, EPS)
    dst = jnp.concatenate(
        [edge_index[1].astype(jnp.int32), jnp.zeros((pad,), jnp.int32)]
    ).reshape(ROWS_TOTAL, EPS)
    normp = jnp.concatenate(
        [norm.astype(jnp.float32), jnp.zeros((pad,), jnp.float32)])
    zeros_nd = jnp.zeros((N_PAD, DE), jnp.float32)

    g0, s0 = _tc_a(z, W_msg0, b_msg0.reshape(1, DE), W_self0)
    g0p = jnp.concatenate([g0, jnp.zeros((N_PAD - N, DE), jnp.float32)])
    agg0p = _edge_pass(src, dst, normp, g0p, zeros_nd)
    g1, s1 = _tc_b(s0, agg0p[:N], agg0p[N_PAD:N_PAD + N], W_agg0, b_node0.reshape(1, D),
                   W_msg1, W_e1, b_msg1.reshape(1, DE), g0, W_self1)
    g1p = jnp.concatenate([g1, jnp.zeros((N_PAD - N, DE), jnp.float32)])
    agg1p = _edge_pass(src, dst, normp, g1p, zeros_nd)
    return _tc_c(s1, agg1p[:N], agg1p[N_PAD:N_PAD + N], W_agg1, b_node1.reshape(1, D))


# speedup vs baseline: 1.0083x; 1.0083x over previous
import functools

import jax
import jax.numpy as jnp
from jax import lax
from jax.experimental import pallas as pl
from jax.experimental.pallas import tpu as pltpu
from jax.experimental.pallas import tpu_sc as plsc

N, E, D, DE = 10000, 320000, 128, 16

NC, NS = 2, 16
NW = NC * NS

EPS = 128
RPG = 8
B_EDGE = EPS * RPG
GROUPS = 10
E_PAD = NW * GROUPS * B_EDGE
ROWS_TOTAL = E_PAD // EPS
ROWS_PER_WORKER = ROWS_TOTAL // NW
N_PAD = 10240
NODE_SLICE = N_PAD // NS

_mesh = plsc.VectorSubcoreMesh(
    core_axis_name="c", subcore_axis_name="s", num_cores=NC, num_subcores=NS
)


@functools.partial(
    pl.kernel,
    out_type=jax.ShapeDtypeStruct((NC * N_PAD, DE), jnp.float32),
    mesh=_mesh,
    scratch_types=[
        pltpu.VMEM((ROWS_PER_WORKER, EPS), jnp.int32),
        pltpu.VMEM((ROWS_PER_WORKER, EPS), jnp.int32),
        pltpu.VMEM((ROWS_PER_WORKER * EPS,), jnp.float32),
        pltpu.VMEM((B_EDGE, DE), jnp.float32),
        pltpu.VMEM((B_EDGE, DE), jnp.float32),
        pltpu.VMEM_SHARED((N_PAD, DE), jnp.float32),
        pltpu.VMEM_SHARED((N_PAD, DE), jnp.float32),
        pltpu.SemaphoreType.DMA,
        pltpu.SemaphoreType.DMA,
        pltpu.SemaphoreType.DMA,
        pltpu.SemaphoreType.DMA,
    ],
    compiler_params=pltpu.CompilerParams(use_tc_tiling_on_sc=False),
)
def _edge_pass(src_hbm, dst_hbm, norm_hbm, g_hbm, zeros_hbm, agg_out,
               idx_s, idx_d, norm_v, rows0, rows1, agg_sh, g_sh,
               sem0, sem1, ssem0, ssem1):
    c = lax.axis_index("c")
    s = lax.axis_index("s")
    wid = s * NC + c

    row0 = wid * ROWS_PER_WORKER
    init_copies = [
        pltpu.async_copy(
            zeros_hbm.at[pl.ds(s * NODE_SLICE, NODE_SLICE)],
            agg_sh.at[pl.ds(s * NODE_SLICE, NODE_SLICE)], sem0),
        pltpu.async_copy(
            g_hbm.at[pl.ds(s * NODE_SLICE, NODE_SLICE)],
            g_sh.at[pl.ds(s * NODE_SLICE, NODE_SLICE)], sem0),
        pltpu.async_copy(src_hbm.at[pl.ds(row0, ROWS_PER_WORKER)],
                         idx_s, sem1),
        pltpu.async_copy(dst_hbm.at[pl.ds(row0, ROWS_PER_WORKER)],
                         idx_d, sem1),
        pltpu.async_copy(norm_hbm.at[pl.ds(row0 * EPS,
                                           ROWS_PER_WORKER * EPS)],
                         norm_v, sem1),
    ]
    for cp in init_copies:
        cp.wait()
    plsc.subcore_barrier()

    bufs = (rows0, rows1)
    gsems = (sem0, sem1)
    ssems = (ssem0, ssem1)

    def stage(b, gi):
        return [
            pltpu.async_copy(g_sh.at[idx_s.at[gi * RPG + j]],
                             bufs[b].at[pl.ds(j * EPS, EPS)], gsems[b])
            for j in range(RPG)
        ]

    inflight = {0: stage(0, 0)}
    scat = {0: [], 1: []}
    for gi in range(GROUPS):
        b = gi & 1
        if gi + 1 < GROUPS:
            for cp in scat[1 - b]:
                cp.wait()
            scat[1 - b] = []
            inflight[1 - b] = stage(1 - b, gi + 1)
        for cp in inflight[b]:
            cp.wait()

        rows_v = bufs[b]
        nbase = gi * B_EDGE

        def chunk_body(m, _, rows_v=rows_v, nbase=nbase):
            nv = norm_v[pl.ds(nbase + m * DE, DE)]
            base = m * DE
            for i in range(DE):
                rows_v[base + i] = rows_v[base + i] * nv[i]
            return 0

        lax.fori_loop(0, B_EDGE // DE, chunk_body, 0, unroll=8)

        scat[b] = [
            pltpu.async_copy(rows_v.at[pl.ds(j * EPS, EPS)],
                             agg_sh.at[idx_d.at[gi * RPG + j]], ssems[b],
                             add=True)
            for j in range(RPG)
        ]

    for b in (0, 1):
        for cp in scat[b]:
            cp.wait()

    plsc.subcore_barrier()
    pltpu.sync_copy(
        agg_sh.at[pl.ds(s * NODE_SLICE, NODE_SLICE)],
        agg_out.at[pl.ds(c * N_PAD + s * NODE_SLICE, NODE_SLICE)],
    )


_ROWS_BLK = 1000
_GRID = N // _ROWS_BLK


def _tc_a_body(z_ref, wm_ref, bm_ref, ws_ref, g0_ref, s0_ref):
    z = z_ref[...]
    g0_ref[...] = jnp.maximum(
        jnp.dot(z, wm_ref[...], preferred_element_type=jnp.float32) + bm_ref[...], 0.0)
    s0_ref[...] = jnp.dot(z, ws_ref[...], preferred_element_type=jnp.float32)


_tc_a = pl.pallas_call(
    _tc_a_body,
    grid=(_GRID,),
    in_specs=[
        pl.BlockSpec((_ROWS_BLK, D), lambda i: (i, 0)),
        pl.BlockSpec((D, DE), lambda i: (0, 0)),
        pl.BlockSpec((1, DE), lambda i: (0, 0)),
        pl.BlockSpec((D, D), lambda i: (0, 0)),
    ],
    out_specs=[
        pl.BlockSpec((_ROWS_BLK, DE), lambda i: (i, 0)),
        pl.BlockSpec((_ROWS_BLK, D), lambda i: (i, 0)),
    ],
    out_shape=[
        jax.ShapeDtypeStruct((N, DE), jnp.float32),
        jax.ShapeDtypeStruct((N, D), jnp.float32),
    ],
)


def _tc_b_body(s0_ref, a0_ref, a1_ref, wagg_ref, bnode_ref, wmsg_ref, we_ref,
               bmsg_ref, g0_ref, wself_ref, g1_ref, s1_ref):
    agg = a0_ref[...] + a1_ref[...]
    x1 = jnp.maximum(
        s0_ref[...]
        + jnp.dot(agg, wagg_ref[...], preferred_element_type=jnp.float32)
        + bnode_ref[...], 0.0)
    g1_ref[...] = jnp.maximum(
        jnp.dot(x1, wmsg_ref[...], preferred_element_type=jnp.float32)
        + jnp.dot(g0_ref[...], we_ref[...], preferred_element_type=jnp.float32)
        + bmsg_ref[...], 0.0)
    s1_ref[...] = jnp.dot(x1, wself_ref[...], preferred_element_type=jnp.float32)


_tc_b = pl.pallas_call(
    _tc_b_body,
    grid=(_GRID,),
    in_specs=[
        pl.BlockSpec((_ROWS_BLK, D), lambda i: (i, 0)),
        pl.BlockSpec((_ROWS_BLK, DE), lambda i: (i, 0)),
        pl.BlockSpec((_ROWS_BLK, DE), lambda i: (i, 0)),
        pl.BlockSpec((DE, D), lambda i: (0, 0)),
        pl.BlockSpec((1, D), lambda i: (0, 0)),
        pl.BlockSpec((D, DE), lambda i: (0, 0)),
        pl.BlockSpec((DE, DE), lambda i: (0, 0)),
        pl.BlockSpec((1, DE), lambda i: (0, 0)),
        pl.BlockSpec((_ROWS_BLK, DE), lambda i: (i, 0)),
        pl.BlockSpec((D, D), lambda i: (0, 0)),
    ],
    out_specs=[
        pl.BlockSpec((_ROWS_BLK, DE), lambda i: (i, 0)),
        pl.BlockSpec((_ROWS_BLK, D), lambda i: (i, 0)),
    ],
    out_shape=[
        jax.ShapeDtypeStruct((N, DE), jnp.float32),
        jax.ShapeDtypeStruct((N, D), jnp.float32),
    ],
)


def _tc_c_body(s1_ref, a0_ref, a1_ref, wagg_ref, bnode_ref, out_ref):
    agg = a0_ref[...] + a1_ref[...]
    out_ref[...] = jnp.maximum(
        s1_ref[...]
        + jnp.dot(agg, wagg_ref[...], preferred_element_type=jnp.float32)
        + bnode_ref[...], 0.0)


_tc_c = pl.pallas_call(
    _tc_c_body,
    grid=(_GRID,),
    in_specs=[
        pl.BlockSpec((_ROWS_BLK, D), lambda i: (i, 0)),
        pl.BlockSpec((_ROWS_BLK, DE), lambda i: (i, 0)),
        pl.BlockSpec((_ROWS_BLK, DE), lambda i: (i, 0)),
        pl.BlockSpec((DE, D), lambda i: (0, 0)),
        pl.BlockSpec((1, D), lambda i: (0, 0)),
    ],
    out_specs=pl.BlockSpec((_ROWS_BLK, D), lambda i: (i, 0)),
    out_shape=jax.ShapeDtypeStruct((N, D), jnp.float32),
)


def kernel(edge_index, z, norm, W_msg0, b_msg0, W_self0, W_agg0, b_node0,
           W_msg1, W_e1, b_msg1, W_self1, W_agg1, b_node1):
    pad = E_PAD - E
    src = jnp.concatenate(
        [edge_index[0].astype(jnp.int32), jnp.zeros((pad,), jnp.int32)]
    ).reshape(ROWS_TOTAL, EPS)
    dst = jnp.concatenate(
        [edge_index[1].astype(jnp.int32), jnp.zeros((pad,), jnp.int32)]
    ).reshape(ROWS_TOTAL, EPS)
    normp = jnp.concatenate(
        [norm.astype(jnp.float32), jnp.zeros((pad,), jnp.float32)])
    zeros_nd = jnp.zeros((N_PAD, DE), jnp.float32)

    g0, s0 = _tc_a(z, W_msg0, b_msg0.reshape(1, DE), W_self0)
    g0p = jnp.concatenate([g0, jnp.zeros((N_PAD - N, DE), jnp.float32)])
    agg0p = _edge_pass(src, dst, normp, g0p, zeros_nd)
    g1, s1 = _tc_b(s0, agg0p[:N], agg0p[N_PAD:N_PAD + N], W_agg0, b_node0.reshape(1, D),
                   W_msg1, W_e1, b_msg1.reshape(1, DE), g0, W_self1)
    g1p = jnp.concatenate([g1, jnp.zeros((N_PAD - N, DE), jnp.float32)])
    agg1p = _edge_pass(src, dst, normp, g1p, zeros_nd)
    return _tc_c(s1, agg1p[:N], agg1p[N_PAD:N_PAD + N], W_agg1, b_node1.reshape(1, D))


# speedup vs baseline: 1.0128x; 1.0045x over previous
import functools

import jax
import jax.numpy as jnp
from jax import lax
from jax.experimental import pallas as pl
from jax.experimental.pallas import tpu as pltpu
from jax.experimental.pallas import tpu_sc as plsc

N, E, D, DE = 10000, 320000, 128, 16

NC, NS = 2, 16
NW = NC * NS

EPS = 128
RPG = 8
B_EDGE = EPS * RPG
GROUPS = 10
E_PAD = NW * GROUPS * B_EDGE
ROWS_TOTAL = E_PAD // EPS
ROWS_PER_WORKER = ROWS_TOTAL // NW
N_PAD = 10240
NODE_SLICE = N_PAD // NS

_mesh = plsc.VectorSubcoreMesh(
    core_axis_name="c", subcore_axis_name="s", num_cores=NC, num_subcores=NS
)


@functools.partial(
    pl.kernel,
    out_type=jax.ShapeDtypeStruct((NC * N_PAD, DE), jnp.float32),
    mesh=_mesh,
    scratch_types=[
        pltpu.VMEM((ROWS_PER_WORKER, EPS), jnp.int32),
        pltpu.VMEM((ROWS_PER_WORKER, EPS), jnp.int32),
        pltpu.VMEM((ROWS_PER_WORKER * EPS,), jnp.float32),
        pltpu.VMEM((B_EDGE, DE), jnp.float32),
        pltpu.VMEM((B_EDGE, DE), jnp.float32),
        pltpu.VMEM_SHARED((N_PAD, DE), jnp.float32),
        pltpu.VMEM_SHARED((N_PAD, DE), jnp.float32),
        pltpu.SemaphoreType.DMA,
        pltpu.SemaphoreType.DMA,
        pltpu.SemaphoreType.DMA,
        pltpu.SemaphoreType.DMA,
    ],
    compiler_params=pltpu.CompilerParams(use_tc_tiling_on_sc=False,
                                         skip_device_barrier=True),
)
def _edge_pass(src_hbm, dst_hbm, norm_hbm, g_hbm, zeros_hbm, agg_out,
               idx_s, idx_d, norm_v, rows0, rows1, agg_sh, g_sh,
               sem0, sem1, ssem0, ssem1):
    c = lax.axis_index("c")
    s = lax.axis_index("s")
    wid = s * NC + c

    row0 = wid * ROWS_PER_WORKER
    init_copies = [
        pltpu.async_copy(
            zeros_hbm.at[pl.ds(s * NODE_SLICE, NODE_SLICE)],
            agg_sh.at[pl.ds(s * NODE_SLICE, NODE_SLICE)], sem0),
        pltpu.async_copy(
            g_hbm.at[pl.ds(s * NODE_SLICE, NODE_SLICE)],
            g_sh.at[pl.ds(s * NODE_SLICE, NODE_SLICE)], sem0),
        pltpu.async_copy(src_hbm.at[pl.ds(row0, ROWS_PER_WORKER)],
                         idx_s, sem1),
        pltpu.async_copy(dst_hbm.at[pl.ds(row0, ROWS_PER_WORKER)],
                         idx_d, sem1),
        pltpu.async_copy(norm_hbm.at[pl.ds(row0 * EPS,
                                           ROWS_PER_WORKER * EPS)],
                         norm_v, sem1),
    ]
    for cp in init_copies:
        cp.wait()
    plsc.subcore_barrier()

    bufs = (rows0, rows1)
    gsems = (sem0, sem1)
    ssems = (ssem0, ssem1)

    def stage(b, gi):
        return [
            pltpu.async_copy(g_sh.at[idx_s.at[gi * RPG + j]],
                             bufs[b].at[pl.ds(j * EPS, EPS)], gsems[b])
            for j in range(RPG)
        ]

    inflight = {0: stage(0, 0)}
    scat = {0: [], 1: []}
    for gi in range(GROUPS):
        b = gi & 1
        if gi + 1 < GROUPS:
            for cp in scat[1 - b]:
                cp.wait()
            scat[1 - b] = []
            inflight[1 - b] = stage(1 - b, gi + 1)
        for cp in inflight[b]:
            cp.wait()

        rows_v = bufs[b]
        nbase = gi * B_EDGE

        def chunk_body(m, _, rows_v=rows_v, nbase=nbase):
            nv = norm_v[pl.ds(nbase + m * DE, DE)]
            base = m * DE
            for i in range(DE):
                rows_v[base + i] = rows_v[base + i] * nv[i]
            return 0

        lax.fori_loop(0, B_EDGE // DE, chunk_body, 0, unroll=4)

        scat[b] = [
            pltpu.async_copy(rows_v.at[pl.ds(j * EPS, EPS)],
                             agg_sh.at[idx_d.at[gi * RPG + j]], ssems[b],
                             add=True)
            for j in range(RPG)
        ]

    for b in (0, 1):
        for cp in scat[b]:
            cp.wait()

    plsc.subcore_barrier()
    pltpu.sync_copy(
        agg_sh.at[pl.ds(s * NODE_SLICE, NODE_SLICE)],
        agg_out.at[pl.ds(c * N_PAD + s * NODE_SLICE, NODE_SLICE)],
    )


_ROWS_BLK = 1000
_GRID = N // _ROWS_BLK


def _tc_a_body(z_ref, wm_ref, bm_ref, ws_ref, g0_ref, s0_ref):
    z = z_ref[...]
    g0_ref[...] = jnp.maximum(
        jnp.dot(z, wm_ref[...], preferred_element_type=jnp.float32) + bm_ref[...], 0.0)
    s0_ref[...] = jnp.dot(z, ws_ref[...], preferred_element_type=jnp.float32)


_tc_a = pl.pallas_call(
    _tc_a_body,
    grid=(_GRID,),
    in_specs=[
        pl.BlockSpec((_ROWS_BLK, D), lambda i: (i, 0)),
        pl.BlockSpec((D, DE), lambda i: (0, 0)),
        pl.BlockSpec((1, DE), lambda i: (0, 0)),
        pl.BlockSpec((D, D), lambda i: (0, 0)),
    ],
    out_specs=[
        pl.BlockSpec((_ROWS_BLK, DE), lambda i: (i, 0)),
        pl.BlockSpec((_ROWS_BLK, D), lambda i: (i, 0)),
    ],
    out_shape=[
        jax.ShapeDtypeStruct((N, DE), jnp.float32),
        jax.ShapeDtypeStruct((N, D), jnp.float32),
    ],
)


def _tc_b_body(s0_ref, a0_ref, a1_ref, wagg_ref, bnode_ref, wmsg_ref, we_ref,
               bmsg_ref, g0_ref, wself_ref, g1_ref, s1_ref):
    agg = a0_ref[...] + a1_ref[...]
    x1 = jnp.maximum(
        s0_ref[...]
        + jnp.dot(agg, wagg_ref[...], preferred_element_type=jnp.float32)
        + bnode_ref[...], 0.0)
    g1_ref[...] = jnp.maximum(
        jnp.dot(x1, wmsg_ref[...], preferred_element_type=jnp.float32)
        + jnp.dot(g0_ref[...], we_ref[...], preferred_element_type=jnp.float32)
        + bmsg_ref[...], 0.0)
    s1_ref[...] = jnp.dot(x1, wself_ref[...], preferred_element_type=jnp.float32)


_tc_b = pl.pallas_call(
    _tc_b_body,
    grid=(_GRID,),
    in_specs=[
        pl.BlockSpec((_ROWS_BLK, D), lambda i: (i, 0)),
        pl.BlockSpec((_ROWS_BLK, DE), lambda i: (i, 0)),
        pl.BlockSpec((_ROWS_BLK, DE), lambda i: (i, 0)),
        pl.BlockSpec((DE, D), lambda i: (0, 0)),
        pl.BlockSpec((1, D), lambda i: (0, 0)),
        pl.BlockSpec((D, DE), lambda i: (0, 0)),
        pl.BlockSpec((DE, DE), lambda i: (0, 0)),
        pl.BlockSpec((1, DE), lambda i: (0, 0)),
        pl.BlockSpec((_ROWS_BLK, DE), lambda i: (i, 0)),
        pl.BlockSpec((D, D), lambda i: (0, 0)),
    ],
    out_specs=[
        pl.BlockSpec((_ROWS_BLK, DE), lambda i: (i, 0)),
        pl.BlockSpec((_ROWS_BLK, D), lambda i: (i, 0)),
    ],
    out_shape=[
        jax.ShapeDtypeStruct((N, DE), jnp.float32),
        jax.ShapeDtypeStruct((N, D), jnp.float32),
    ],
)


def _tc_c_body(s1_ref, a0_ref, a1_ref, wagg_ref, bnode_ref, out_ref):
    agg = a0_ref[...] + a1_ref[...]
    out_ref[...] = jnp.maximum(
        s1_ref[...]
        + jnp.dot(agg, wagg_ref[...], preferred_element_type=jnp.float32)
        + bnode_ref[...], 0.0)


_tc_c = pl.pallas_call(
    _tc_c_body,
    grid=(_GRID,),
    in_specs=[
        pl.BlockSpec((_ROWS_BLK, D), lambda i: (i, 0)),
        pl.BlockSpec((_ROWS_BLK, DE), lambda i: (i, 0)),
        pl.BlockSpec((_ROWS_BLK, DE), lambda i: (i, 0)),
        pl.BlockSpec((DE, D), lambda i: (0, 0)),
        pl.BlockSpec((1, D), lambda i: (0, 0)),
    ],
    out_specs=pl.BlockSpec((_ROWS_BLK, D), lambda i: (i, 0)),
    out_shape=jax.ShapeDtypeStruct((N, D), jnp.float32),
)


def kernel(edge_index, z, norm, W_msg0, b_msg0, W_self0, W_agg0, b_node0,
           W_msg1, W_e1, b_msg1, W_self1, W_agg1, b_node1):
    pad = E_PAD - E
    src = jnp.concatenate(
        [edge_index[0].astype(jnp.int32), jnp.zeros((pad,), jnp.int32)]
    ).reshape(ROWS_TOTAL, EPS)
    dst = jnp.concatenate(
        [edge_index[1].astype(jnp.int32), jnp.zeros((pad,), jnp.int32)]
    ).reshape(ROWS_TOTAL, EPS)
    normp = jnp.concatenate(
        [norm.astype(jnp.float32), jnp.zeros((pad,), jnp.float32)])
    zeros_nd = jnp.zeros((N_PAD, DE), jnp.float32)

    g0, s0 = _tc_a(z, W_msg0, b_msg0.reshape(1, DE), W_self0)
    g0p = jnp.concatenate([g0, jnp.zeros((N_PAD - N, DE), jnp.float32)])
    agg0p = _edge_pass(src, dst, normp, g0p, zeros_nd)
    g1, s1 = _tc_b(s0, agg0p[:N], agg0p[N_PAD:N_PAD + N], W_agg0, b_node0.reshape(1, D),
                   W_msg1, W_e1, b_msg1.reshape(1, DE), g0, W_self1)
    g1p = jnp.concatenate([g1, jnp.zeros((N_PAD - N, DE), jnp.float32)])
    agg1p = _edge_pass(src, dst, normp, g1p, zeros_nd)
    return _tc_c(s1, agg1p[:N], agg1p[N_PAD:N_PAD + N], W_agg1, b_node1.reshape(1, D))


# speedup vs baseline: 1.0483x; 1.0350x over previous
import functools

import jax
import jax.numpy as jnp
from jax import lax
from jax.experimental import pallas as pl
from jax.experimental.pallas import tpu as pltpu
from jax.experimental.pallas import tpu_sc as plsc

N, E, D, DE = 10000, 320000, 128, 16

NC, NS = 2, 16
NW = NC * NS

EPS = 128
RPG = 8
B_EDGE = EPS * RPG
ROWS_TOTAL = E // EPS
ROWS_BASE = ROWS_TOTAL // NW
ROWS_MAX = ROWS_BASE + 1
EXTRA_W0 = NW - (ROWS_TOTAL - ROWS_BASE * NW)
GROUP_SIZES = [8] * 9 + [6]
N_PAD = 10240
NODE_SLICE = N_PAD // NS

_mesh = plsc.VectorSubcoreMesh(
    core_axis_name="c", subcore_axis_name="s", num_cores=NC, num_subcores=NS
)


@functools.partial(
    pl.kernel,
    out_type=jax.ShapeDtypeStruct((NC * N_PAD, DE), jnp.float32),
    mesh=_mesh,
    scratch_types=[
        pltpu.VMEM((ROWS_MAX, EPS), jnp.int32),
        pltpu.VMEM((ROWS_MAX, EPS), jnp.int32),
        pltpu.VMEM((ROWS_MAX * EPS,), jnp.float32),
        pltpu.VMEM((B_EDGE, DE), jnp.float32),
        pltpu.VMEM((B_EDGE, DE), jnp.float32),
        pltpu.VMEM_SHARED((N_PAD, DE), jnp.float32),
        pltpu.VMEM_SHARED((N_PAD, DE), jnp.float32),
        pltpu.SemaphoreType.DMA,
        pltpu.SemaphoreType.DMA,
        pltpu.SemaphoreType.DMA,
        pltpu.SemaphoreType.DMA,
    ],
    compiler_params=pltpu.CompilerParams(use_tc_tiling_on_sc=False),
)
def _edge_pass(src_hbm, dst_hbm, norm_hbm, g_hbm, zeros_hbm, agg_out,
               idx_s, idx_d, norm_v, rows0, rows1, agg_sh, g_sh,
               sem0, sem1, ssem0, ssem1):
    c = lax.axis_index("c")
    s = lax.axis_index("s")
    wid = s * NC + c

    row0 = ROWS_BASE * wid + jnp.maximum(wid - EXTRA_W0, 0)
    init_copies = [
        pltpu.async_copy(
            zeros_hbm.at[pl.ds(s * NODE_SLICE, NODE_SLICE)],
            agg_sh.at[pl.ds(s * NODE_SLICE, NODE_SLICE)], sem0),
        pltpu.async_copy(
            g_hbm.at[pl.ds(s * NODE_SLICE, NODE_SLICE)],
            g_sh.at[pl.ds(s * NODE_SLICE, NODE_SLICE)], sem0),
        pltpu.async_copy(src_hbm.at[pl.ds(row0, ROWS_MAX)], idx_s, sem1),
        pltpu.async_copy(dst_hbm.at[pl.ds(row0, ROWS_MAX)], idx_d, sem1),
        pltpu.async_copy(norm_hbm.at[pl.ds(row0 * EPS, ROWS_MAX * EPS)],
                         norm_v, sem1),
    ]
    for cp in init_copies:
        cp.wait()
    plsc.subcore_barrier()

    bufs = (rows0, rows1)
    gsems = (sem0, sem1)
    ssems = (ssem0, ssem1)
    row_off = [sum(GROUP_SIZES[:i]) for i in range(len(GROUP_SIZES))]

    def stage(b, gi):
        return [
            pltpu.async_copy(g_sh.at[idx_s.at[row_off[gi] + j]],
                             bufs[b].at[pl.ds(j * EPS, EPS)], gsems[b])
            for j in range(GROUP_SIZES[gi])
        ]

    def scale(rows_v, nbase, nchunks):
        def chunk_body(m, _):
            nv = norm_v[pl.ds(nbase + m * DE, DE)]
            base = m * DE
            for i in range(DE):
                rows_v[base + i] = rows_v[base + i] * nv[i]
            return 0

        lax.fori_loop(0, nchunks, chunk_body, 0, unroll=4)

    n_groups = len(GROUP_SIZES)
    inflight = {0: stage(0, 0)}
    scat = {0: [], 1: []}
    for gi in range(n_groups):
        b = gi & 1
        if gi + 1 < n_groups:
            for cp in scat[1 - b]:
                cp.wait()
            scat[1 - b] = []
            inflight[1 - b] = stage(1 - b, gi + 1)
        for cp in inflight[b]:
            cp.wait()

        rows_v = bufs[b]
        scale(rows_v, row_off[gi] * EPS, GROUP_SIZES[gi] * EPS // DE)

        scat[b] = [
            pltpu.async_copy(rows_v.at[pl.ds(j * EPS, EPS)],
                             agg_sh.at[idx_d.at[row_off[gi] + j]], ssems[b],
                             add=True)
            for j in range(GROUP_SIZES[gi])
        ]

    for b in (0, 1):
        for cp in scat[b]:
            cp.wait()

    @pl.when(wid >= EXTRA_W0)
    def _tail():
        pltpu.async_copy(g_sh.at[idx_s.at[ROWS_BASE]],
                         rows0.at[pl.ds(0, EPS)], sem0).wait()
        scale(rows0, ROWS_BASE * EPS, EPS // DE)
        pltpu.async_copy(rows0.at[pl.ds(0, EPS)],
                         agg_sh.at[idx_d.at[ROWS_BASE]], ssem0,
                         add=True).wait()

    plsc.subcore_barrier()
    pltpu.sync_copy(
        agg_sh.at[pl.ds(s * NODE_SLICE, NODE_SLICE)],
        agg_out.at[pl.ds(c * N_PAD + s * NODE_SLICE, NODE_SLICE)],
    )


_ROWS_BLK = 1000
_GRID = N // _ROWS_BLK


def _tc_a_body(z_ref, wm_ref, bm_ref, ws_ref, g0_ref, s0_ref):
    z = z_ref[...]
    g0_ref[...] = jnp.maximum(
        jnp.dot(z, wm_ref[...], preferred_element_type=jnp.float32) + bm_ref[...], 0.0)
    s0_ref[...] = jnp.dot(z, ws_ref[...], preferred_element_type=jnp.float32)


_tc_a = pl.pallas_call(
    _tc_a_body,
    grid=(_GRID,),
    in_specs=[
        pl.BlockSpec((_ROWS_BLK, D), lambda i: (i, 0)),
        pl.BlockSpec((D, DE), lambda i: (0, 0)),
        pl.BlockSpec((1, DE), lambda i: (0, 0)),
        pl.BlockSpec((D, D), lambda i: (0, 0)),
    ],
    out_specs=[
        pl.BlockSpec((_ROWS_BLK, DE), lambda i: (i, 0)),
        pl.BlockSpec((_ROWS_BLK, D), lambda i: (i, 0)),
    ],
    out_shape=[
        jax.ShapeDtypeStruct((N_PAD, DE), jnp.float32),
        jax.ShapeDtypeStruct((N, D), jnp.float32),
    ],
)


def _tc_b_body(s0_ref, a0_ref, a1_ref, wagg_ref, bnode_ref, wmsg_ref, we_ref,
               bmsg_ref, g0_ref, wself_ref, g1_ref, s1_ref):
    agg = a0_ref[...] + a1_ref[...]
    x1 = jnp.maximum(
        s0_ref[...]
        + jnp.dot(agg, wagg_ref[...], preferred_element_type=jnp.float32)
        + bnode_ref[...], 0.0)
    g1_ref[...] = jnp.maximum(
        jnp.dot(x1, wmsg_ref[...], preferred_element_type=jnp.float32)
        + jnp.dot(g0_ref[...], we_ref[...], preferred_element_type=jnp.float32)
        + bmsg_ref[...], 0.0)
    s1_ref[...] = jnp.dot(x1, wself_ref[...], preferred_element_type=jnp.float32)


_tc_b = pl.pallas_call(
    _tc_b_body,
    grid=(_GRID,),
    in_specs=[
        pl.BlockSpec((_ROWS_BLK, D), lambda i: (i, 0)),
        pl.BlockSpec((_ROWS_BLK, DE), lambda i: (i, 0)),
        pl.BlockSpec((_ROWS_BLK, DE), lambda i: (i, 0)),
        pl.BlockSpec((DE, D), lambda i: (0, 0)),
        pl.BlockSpec((1, D), lambda i: (0, 0)),
        pl.BlockSpec((D, DE), lambda i: (0, 0)),
        pl.BlockSpec((DE, DE), lambda i: (0, 0)),
        pl.BlockSpec((1, DE), lambda i: (0, 0)),
        pl.BlockSpec((_ROWS_BLK, DE), lambda i: (i, 0)),
        pl.BlockSpec((D, D), lambda i: (0, 0)),
    ],
    out_specs=[
        pl.BlockSpec((_ROWS_BLK, DE), lambda i: (i, 0)),
        pl.BlockSpec((_ROWS_BLK, D), lambda i: (i, 0)),
    ],
    out_shape=[
        jax.ShapeDtypeStruct((N_PAD, DE), jnp.float32),
        jax.ShapeDtypeStruct((N, D), jnp.float32),
    ],
)


def _tc_c_body(s1_ref, a0_ref, a1_ref, wagg_ref, bnode_ref, out_ref):
    agg = a0_ref[...] + a1_ref[...]
    out_ref[...] = jnp.maximum(
        s1_ref[...]
        + jnp.dot(agg, wagg_ref[...], preferred_element_type=jnp.float32)
        + bnode_ref[...], 0.0)


_tc_c = pl.pallas_call(
    _tc_c_body,
    grid=(_GRID,),
    in_specs=[
        pl.BlockSpec((_ROWS_BLK, D), lambda i: (i, 0)),
        pl.BlockSpec((_ROWS_BLK, DE), lambda i: (i, 0)),
        pl.BlockSpec((_ROWS_BLK, DE), lambda i: (i, 0)),
        pl.BlockSpec((DE, D), lambda i: (0, 0)),
        pl.BlockSpec((1, D), lambda i: (0, 0)),
    ],
    out_specs=pl.BlockSpec((_ROWS_BLK, D), lambda i: (i, 0)),
    out_shape=jax.ShapeDtypeStruct((N, D), jnp.float32),
)


def kernel(edge_index, z, norm, W_msg0, b_msg0, W_self0, W_agg0, b_node0,
           W_msg1, W_e1, b_msg1, W_self1, W_agg1, b_node1):
    src = edge_index[0].astype(jnp.int32).reshape(ROWS_TOTAL, EPS)
    dst = edge_index[1].astype(jnp.int32).reshape(ROWS_TOTAL, EPS)
    normf = norm.astype(jnp.float32)
    zeros_nd = jnp.zeros((N_PAD, DE), jnp.float32)

    g0, s0 = _tc_a(z, W_msg0, b_msg0.reshape(1, DE), W_self0)
    agg0p = _edge_pass(src, dst, normf, g0, zeros_nd)
    g1, s1 = _tc_b(s0, agg0p[:N], agg0p[N_PAD:N_PAD + N], W_agg0, b_node0.reshape(1, D),
                   W_msg1, W_e1, b_msg1.reshape(1, DE), g0, W_self1)
    agg1p = _edge_pass(src, dst, normf, g1, zeros_nd)
    return _tc_c(s1, agg1p[:N], agg1p[N_PAD:N_PAD + N], W_agg1, b_node1.reshape(1, D))
